# baseline (speedup 1.0000x reference)
import functools

import jax
import jax.numpy as jnp
from jax import lax
from jax.experimental import pallas as pl
from jax.experimental.pallas import tpu as pltpu
from jax.experimental.pallas import tpu_sc as plsc

N = 10000
E = 320000
D = 128
NUM_RELS = 8
NUM_BASES = 4

NC = 2
NS = 16
NW = NC * NS

C = 80
NCH = 128
EW = NCH * C
E_PAD = NW * EW
TOTCH = E_PAD // C
NB = 4
NROWS = 624
NTAIL = N - NS * NROWS

BN = 2000
NBLK = N // BN

_HIGHEST = jax.lax.Precision.HIGHEST


def _wcat_body(wc_ref, wf_ref, lw_ref, o_ref):
    o_ref[0:NUM_RELS, :] = jax.lax.dot_general(
        wc_ref[...], wf_ref[...],
        dimension_numbers=(((1,), (0,)), ((), ())),
        preferred_element_type=jnp.float32, precision=_HIGHEST)
    o_ref[NUM_RELS:NUM_RELS + 1, :] = lw_ref[...]


def _y_body(x_ref, w_ref, o_ref):
    o_ref[...] = jnp.dot(x_ref[...], w_ref[0],
                         preferred_element_type=jnp.float32)


def _combine_body(a_ref, b_ref, c_ref, bias_ref, o_ref):
    o_ref[...] = a_ref[...] + b_ref[...] + c_ref[...] + bias_ref[...]


def _sc_edge_body(fields_hbm, ycat_hbm,
                  out_hbm,
                  fields_v, idx2d, dst2d, rows_v, acc,
                  semf0, semf1, semf2, semf3, semg0, semg1, semg2, semg3,
                  sems0, sems1, sems2, sems3):
    c = lax.axis_index("c")
    s = lax.axis_index("s")
    semf = (semf0, semf1, semf2, semf3)
    semg = (semg0, semg1, semg2, semg3)
    sems = (sems0, sems1, sems2, sems3)

    def z_body(i, carry):
        for j in range(D // 16):
            rows_v[0, i, pl.ds(j * 16, 16)] = jnp.zeros((16,), jnp.float32)
        return carry

    lax.fori_loop(0, C, z_body, 0)
    row0 = s * NROWS
    for t in range(NROWS // C):
        pltpu.sync_copy(rows_v.at[0], acc.at[pl.ds(row0 + t * C, C)])
    pltpu.sync_copy(rows_v.at[0, pl.ds(0, NROWS % C)],
                    acc.at[pl.ds(row0 + (NROWS // C) * C, NROWS % C)])

    @pl.when(s == 0)
    def _zero_tail():
        pltpu.sync_copy(rows_v.at[0, pl.ds(0, NTAIL)],
                        acc.at[pl.ds(NS * NROWS, NTAIL)])

    plsc.subcore_barrier()

    wid = c * NS + s
    ch0 = wid * NCH

    def issue_fields(k, b):
        pltpu.async_copy(fields_hbm.at[ch0 + k], fields_v.at[b], semf[b])

    def wait_fields(b):
        pltpu.make_async_copy(fields_hbm.at[0], fields_v.at[b],
                              semf[b]).wait()

    def pack_indices(b):
        for q in range(C // 16):
            sl = pl.ds(q * 16, 16)
            et16 = fields_v[b, 0, sl]
            s16 = fields_v[b, 1, sl]
            idx2d[b, 0, sl] = (et16 * N + s16).astype(jnp.int32)
            dst2d[b, 0, sl] = fields_v[b, 2, sl].astype(jnp.int32)

    def issue_gather(b):
        pltpu.async_copy(ycat_hbm.at[idx2d.at[b, 0]], rows_v.at[b], semg[b])

    def wait_gather(b):
        pltpu.make_async_copy(ycat_hbm.at[idx2d.at[b, 0]], rows_v.at[b],
                              semg[b]).wait()

    def issue_scatter(b):
        pltpu.async_copy(rows_v.at[b], acc.at[dst2d.at[b, 0]], sems[b],
                         add=True)

    def wait_scatter(b):
        pltpu.make_async_copy(rows_v.at[b], acc.at[dst2d.at[b, 0]],
                              sems[b]).wait()

    def scale(b):
        def g_body(g, carry2):
            n16 = fields_v[b, 3, pl.ds(g * 16, 16)]
            e0 = g * 16
            for k in range(16):
                nv = jnp.full((16,), n16[k])
                for j in range(D // 16):
                    sl = pl.ds(j * 16, 16)
                    rows_v[b, e0 + k, sl] = rows_v[b, e0 + k, sl] * nv
            return carry2

        lax.fori_loop(0, C // 16, g_body, 0)

    LEAD = NB - 1
    for i in range(NB):
        issue_fields(i, i)
    for i in range(LEAD):
        wait_fields(i)
        pack_indices(i)
        issue_gather(i)

    def ring_body(kk, carry):
        for b in range(NB):
            k = NB * kk + b
            nb = (b + LEAD) % NB

            @pl.when(k + LEAD < NCH)
            def _advance():
                wait_fields(nb)

                @pl.when(k >= 1)
                def _retire_prev_scatter():
                    wait_scatter(nb)

                pack_indices(nb)
                issue_gather(nb)

            wait_gather(b)
            scale(b)

            @pl.when(k + NB < NCH)
            def _prefetch_fields():
                issue_fields(k + NB, b)

            issue_scatter(b)
        return carry

    lax.fori_loop(0, NCH // NB, ring_body, 0)
    for i in range(NB):
        wait_scatter(i)
    plsc.subcore_barrier()

    pltpu.sync_copy(acc.at[pl.ds(row0, NROWS)],
                    out_hbm.at[pl.ds(c * N + row0, NROWS)])

    @pl.when(s == 0)
    def _copy_tail():
        pltpu.sync_copy(acc.at[pl.ds(NS * NROWS, NTAIL)],
                        out_hbm.at[pl.ds(c * N + NS * NROWS, NTAIL)])


@functools.cache
def _sc_edge_kernel_fn():
    mesh = plsc.VectorSubcoreMesh(core_axis_name="c", subcore_axis_name="s",
                                  num_cores=NC, num_subcores=NS)
    return pl.kernel(
        _sc_edge_body,
        out_type=jax.ShapeDtypeStruct((NC * N, D), jnp.float32),
        mesh=mesh,
        scratch_types=[
            pltpu.VMEM((NB, 4, C), jnp.float32),
            pltpu.VMEM((NB, 1, C), jnp.int32),
            pltpu.VMEM((NB, 1, C), jnp.int32),
            pltpu.VMEM((NB, C, D), jnp.float32),
            pltpu.VMEM_SHARED((N, D), jnp.float32),
        ] + [pltpu.SemaphoreType.DMA] * 12,
    )


@jax.jit
def kernel(feat, edge_index, etypes, norm, weight, w_comp, h_bias, loop_weight):
    src = edge_index[0]
    dst = edge_index[1]
    normf = norm.reshape(E)

    pad = E_PAD - E
    src_p = jnp.pad(src, (0, pad))
    dst_p = jnp.pad(dst, (0, pad))
    et_p = jnp.pad(etypes, (0, pad))
    norm_p = jnp.pad(normf, (0, pad))

    fields = jnp.stack(
        [et_p.astype(jnp.float32), src_p.astype(jnp.float32),
         dst_p.astype(jnp.float32), norm_p],
        axis=0)
    fields = fields.reshape(4, TOTCH, C).transpose(1, 0, 2)

    wcat = pl.pallas_call(
        _wcat_body,
        out_shape=jax.ShapeDtypeStruct((NUM_RELS + 1, D * D), jnp.float32),
    )(w_comp, weight.reshape(NUM_BASES, D * D), loop_weight.reshape(1, D * D))
    wcat3 = wcat.reshape(NUM_RELS + 1, D, D)

    ycat = pl.pallas_call(
        _y_body,
        grid=(NBLK, NUM_RELS + 1),
        in_specs=[
            pl.BlockSpec((BN, D), lambda i, r: (i, 0)),
            pl.BlockSpec((1, D, D), lambda i, r: (r, 0, 0)),
        ],
        out_specs=pl.BlockSpec((BN, D), lambda i, r: (r * NBLK + i, 0)),
        out_shape=jax.ShapeDtypeStruct(((NUM_RELS + 1) * N, D), jnp.float32),
    )(feat, wcat3)

    planes = _sc_edge_kernel_fn()(fields, ycat)

    out = pl.pallas_call(
        _combine_body,
        grid=(NBLK,),
        in_specs=[
            pl.BlockSpec((BN, D), lambda i: (i, 0)),
            pl.BlockSpec((BN, D), lambda i: (i + NBLK, 0)),
            pl.BlockSpec((BN, D), lambda i: (NUM_RELS * NBLK + i, 0)),
            pl.BlockSpec((1, D), lambda i: (0, 0)),
        ],
        out_specs=pl.BlockSpec((BN, D), lambda i: (i, 0)),
        out_shape=jax.ShapeDtypeStruct((N, D), jnp.float32),
    )(planes, planes, ycat, h_bias.reshape(1, D))
    return out


# speedup vs baseline: 1.7321x; 1.7321x over previous
import functools

import jax
import jax.numpy as jnp
from jax import lax
from jax.experimental import pallas as pl
from jax.experimental.pallas import tpu as pltpu
from jax.experimental.pallas import tpu_sc as plsc

N = 10000
E = 320000
D = 128
NUM_RELS = 8
NUM_BASES = 4

NC = 2
NS = 16
NW = NC * NS

C = 112
NCH = 90
EW = NCH * C
E_PAD = NW * EW
TOTCH = E_PAD // C
NB = 3
NROWS = 624
NTAIL = N - NS * NROWS

BN = 2000
NBLK = N // BN

_HIGHEST = jax.lax.Precision.HIGHEST


def _wcat_body(wc_ref, wf_ref, lw_ref, o_ref):
    o_ref[0:NUM_RELS, :] = jax.lax.dot_general(
        wc_ref[...], wf_ref[...],
        dimension_numbers=(((1,), (0,)), ((), ())),
        preferred_element_type=jnp.float32, precision=_HIGHEST)
    o_ref[NUM_RELS:NUM_RELS + 1, :] = lw_ref[...]


def _y_body(x_ref, w_ref, o_ref):
    o_ref[...] = jnp.dot(x_ref[...], w_ref[0],
                         preferred_element_type=jnp.float32)


def _combine_body(a_ref, b_ref, c_ref, bias_ref, o_ref):
    o_ref[...] = a_ref[...] + b_ref[...] + c_ref[...] + bias_ref[...]


def _sc_edge_body(fields_hbm, ycat_hbm,
                  out_hbm,
                  fields_v, idx2d, dst2d, rows_v, acc,
                  semf0, semf1, semf2, semg0, semg1, semg2,
                  sems0, sems1, sems2):
    c = lax.axis_index("c")
    s = lax.axis_index("s")
    semf = (semf0, semf1, semf2)
    semg = (semg0, semg1, semg2)
    sems = (sems0, sems1, sems2)

    def z_body(i, carry):
        for j in range(D // 16):
            rows_v[0, i, pl.ds(j * 16, 16)] = jnp.zeros((16,), jnp.float32)
        return carry

    lax.fori_loop(0, C, z_body, 0)
    row0 = s * NROWS
    for t in range(NROWS // C):
        pltpu.sync_copy(rows_v.at[0], acc.at[pl.ds(row0 + t * C, C)])
    pltpu.sync_copy(rows_v.at[0, pl.ds(0, NROWS % C)],
                    acc.at[pl.ds(row0 + (NROWS // C) * C, NROWS % C)])

    @pl.when(s == 0)
    def _zero_tail():
        pltpu.sync_copy(rows_v.at[0, pl.ds(0, NTAIL)],
                        acc.at[pl.ds(NS * NROWS, NTAIL)])

    plsc.subcore_barrier()

    wid = c * NS + s
    ch0 = wid * NCH

    def issue_fields(k, b):
        pltpu.async_copy(fields_hbm.at[ch0 + k], fields_v.at[b], semf[b])

    def wait_fields(b):
        pltpu.make_async_copy(fields_hbm.at[0], fields_v.at[b],
                              semf[b]).wait()

    def pack_indices(b):
        for q in range(C // 16):
            sl = pl.ds(q * 16, 16)
            et16 = fields_v[b, 0, sl]
            s16 = fields_v[b, 1, sl]
            idx2d[b, 0, sl] = (et16 * N + s16).astype(jnp.int32)
            dst2d[b, 0, sl] = fields_v[b, 2, sl].astype(jnp.int32)

    def issue_gather(b):
        pltpu.async_copy(ycat_hbm.at[idx2d.at[b, 0]], rows_v.at[b], semg[b])

    def wait_gather(b):
        pltpu.make_async_copy(ycat_hbm.at[idx2d.at[b, 0]], rows_v.at[b],
                              semg[b]).wait()

    def issue_scatter(b):
        pltpu.async_copy(rows_v.at[b], acc.at[dst2d.at[b, 0]], sems[b],
                         add=True)

    def wait_scatter(b):
        pltpu.make_async_copy(rows_v.at[b], acc.at[dst2d.at[b, 0]],
                              sems[b]).wait()

    def scale(b):
        def g_body(g, carry2):
            n16 = fields_v[b, 3, pl.ds(g * 16, 16)]
            e0 = g * 16
            for k in range(16):
                nv = jnp.full((16,), n16[k])
                for j in range(D // 16):
                    sl = pl.ds(j * 16, 16)
                    rows_v[b, e0 + k, sl] = rows_v[b, e0 + k, sl] * nv
            return carry2

        lax.fori_loop(0, C // 16, g_body, 0)

    issue_fields(0, 0)
    issue_fields(1, 1)
    issue_fields(2, 2)
    wait_fields(0)
    pack_indices(0)
    issue_gather(0)
    wait_fields(1)
    pack_indices(1)
    issue_gather(1)

    def ring_body(kk, carry):
        for b in range(NB):
            k = NB * kk + b
            nb = (b + 2) % NB

            @pl.when(k + 2 < NCH)
            def _advance():
                wait_fields(nb)

                @pl.when(k >= 1)
                def _retire_prev_scatter():
                    wait_scatter(nb)

                pack_indices(nb)
                issue_gather(nb)

            wait_gather(b)
            scale(b)

            @pl.when(k + 3 < NCH)
            def _prefetch_fields():
                issue_fields(k + 3, b)

            issue_scatter(b)
        return carry

    lax.fori_loop(0, NCH // NB, ring_body, 0)
    wait_scatter(0)
    wait_scatter(1)
    wait_scatter(2)
    plsc.subcore_barrier()

    pltpu.sync_copy(acc.at[pl.ds(row0, NROWS)],
                    out_hbm.at[pl.ds(c * N + row0, NROWS)])

    @pl.when(s == 0)
    def _copy_tail():
        pltpu.sync_copy(acc.at[pl.ds(NS * NROWS, NTAIL)],
                        out_hbm.at[pl.ds(c * N + NS * NROWS, NTAIL)])


@functools.cache
def _sc_edge_kernel_fn():
    mesh = plsc.VectorSubcoreMesh(core_axis_name="c", subcore_axis_name="s",
                                  num_cores=NC, num_subcores=NS)
    return pl.kernel(
        _sc_edge_body,
        out_type=jax.ShapeDtypeStruct((NC * N, D), jnp.float32),
        mesh=mesh,
        scratch_types=[
            pltpu.VMEM((NB, 4, C), jnp.float32),
            pltpu.VMEM((NB, 1, C), jnp.int32),
            pltpu.VMEM((NB, 1, C), jnp.int32),
            pltpu.VMEM((NB, C, D), jnp.float32),
            pltpu.VMEM_SHARED((N, D), jnp.float32),
        ] + [pltpu.SemaphoreType.DMA] * 9,
    )


@jax.jit
def kernel(feat, edge_index, etypes, norm, weight, w_comp, h_bias, loop_weight):
    src = edge_index[0]
    dst = edge_index[1]
    normf = norm.reshape(E)

    pad = E_PAD - E
    src_p = jnp.pad(src, (0, pad))
    dst_p = jnp.pad(dst, (0, pad))
    et_p = jnp.pad(etypes, (0, pad))
    norm_p = jnp.pad(normf, (0, pad))

    fields = jnp.stack(
        [et_p.astype(jnp.float32), src_p.astype(jnp.float32),
         dst_p.astype(jnp.float32), norm_p],
        axis=0)
    fields = fields.reshape(4, TOTCH, C).transpose(1, 0, 2)

    wcat = pl.pallas_call(
        _wcat_body,
        out_shape=jax.ShapeDtypeStruct((NUM_RELS + 1, D * D), jnp.float32),
    )(w_comp, weight.reshape(NUM_BASES, D * D), loop_weight.reshape(1, D * D))
    wcat3 = wcat.reshape(NUM_RELS + 1, D, D)

    ycat = pl.pallas_call(
        _y_body,
        grid=(NBLK, NUM_RELS + 1),
        in_specs=[
            pl.BlockSpec((BN, D), lambda i, r: (i, 0)),
            pl.BlockSpec((1, D, D), lambda i, r: (r, 0, 0)),
        ],
        out_specs=pl.BlockSpec((BN, D), lambda i, r: (r * NBLK + i, 0)),
        out_shape=jax.ShapeDtypeStruct(((NUM_RELS + 1) * N, D), jnp.float32),
    )(feat, wcat3)

    planes = _sc_edge_kernel_fn()(fields, ycat)

    out = pl.pallas_call(
        _combine_body,
        grid=(NBLK,),
        in_specs=[
            pl.BlockSpec((BN, D), lambda i: (i, 0)),
            pl.BlockSpec((BN, D), lambda i: (i + NBLK, 0)),
            pl.BlockSpec((BN, D), lambda i: (NUM_RELS * NBLK + i, 0)),
            pl.BlockSpec((1, D), lambda i: (0, 0)),
        ],
        out_specs=pl.BlockSpec((BN, D), lambda i: (i, 0)),
        out_shape=jax.ShapeDtypeStruct((N, D), jnp.float32),
    )(planes, planes, ycat, h_bias.reshape(1, D))
    return out
